# baseline (speedup 1.0000x reference)
import functools
import math

import jax
import jax.numpy as jnp
from jax.experimental import pallas as pl
from jax.experimental.pallas import tpu as pltpu

_EPS = 1e-6


def _gem_dense_kernel(p_ref, x_ref, a_ref, o_ref, *, inv_s):
    p = p_ref[0]
    x = jnp.maximum(x_ref[...], _EPS)
    xp = jnp.exp2(p * jnp.log2(x))
    z = jnp.dot(xp.astype(jnp.bfloat16), a_ref[...],
                preferred_element_type=jnp.float32)
    m = z * inv_s
    o_ref[...] = jnp.exp2(jnp.log2(m) / p).astype(o_ref.dtype)


def kernel(x, p):
    N, C, H, W = x.shape
    S = H * W
    total = N * C * S
    g = math.gcd(S, 128)
    chunk = S * (128 // g)
    segs = chunk // S
    assert total % chunk == 0, "row count must tile the dense chunk layout"
    M = total // chunk

    xf = x.reshape(M, chunk)
    p_arr = jnp.asarray(p, dtype=jnp.float32).reshape((1,))
    a = (jnp.arange(chunk, dtype=jnp.int32)[:, None] // S
         == jnp.arange(segs, dtype=jnp.int32)[None, :]).astype(jnp.bfloat16)

    tile_m = min(256, M)
    n_m = pl.cdiv(M, tile_m)

    out = pl.pallas_call(
        functools.partial(_gem_dense_kernel, inv_s=1.0 / S),
        out_shape=jax.ShapeDtypeStruct((M, segs), x.dtype),
        grid=(n_m,),
        in_specs=[
            pl.BlockSpec(memory_space=pltpu.MemorySpace.SMEM),
            pl.BlockSpec((tile_m, chunk), lambda i: (i, 0)),
            pl.BlockSpec((chunk, segs), lambda i: (0, 0)),
        ],
        out_specs=pl.BlockSpec((tile_m, segs), lambda i: (i, 0)),
        compiler_params=pltpu.CompilerParams(
            dimension_semantics=("parallel",)),
    )(p_arr, xf, a)

    return out.reshape(N, C)
